# baseline (speedup 1.0000x reference)
import functools

import jax
import jax.numpy as jnp
from jax import lax
from jax.experimental import pallas as pl
from jax.experimental.pallas import tpu as pltpu
from jax.experimental.pallas import tpu_sc as plsc

_F = 26
_D = 32
_B = 4096
_VOCAB = 100000
_N = _B * _F

_NW = 32
_BPW = _N // _NW
_CH = 128
_NCH = _BPW // _CH
_RING = 2

_BLK = 1024

_mesh = plsc.VectorSubcoreMesh(core_axis_name="c", subcore_axis_name="s")


@functools.partial(
    pl.kernel,
    mesh=_mesh,
    out_type=jax.ShapeDtypeStruct((_N, 4 * _D), jnp.float32),
    scratch_types=[
        pltpu.VMEM((_NCH, _CH), jnp.int32),
        pltpu.VMEM((_CH, _CH), jnp.float32),
        pltpu.VMEM((_CH, _CH), jnp.float32),
        pltpu.SemaphoreType.DMA((_RING,)),
    ],
)
def _sc_gather(tab4, gidx_hbm, slab_out, gidx_v, slab0, slab1, sems):
    slabs = [slab0, slab1]
    w = lax.axis_index("s") * 2 + lax.axis_index("c")
    nbase = w * _BPW
    pltpu.sync_copy(gidx_hbm.at[w], gidx_v)

    for b in range(_RING):
        pltpu.async_copy(tab4.at[gidx_v.at[b]], slabs[b], sems.at[b])

    def _group(g, carry):
        for b in range(_RING):
            cb = g * _RING + b
            pltpu.make_async_copy(
                tab4.at[gidx_v.at[0]], slabs[b], sems.at[b]
            ).wait()
            pltpu.sync_copy(
                slabs[b], slab_out.at[pl.ds(nbase + cb * _CH, _CH)]
            )

            @pl.when(cb + _RING < _NCH)
            def _():
                pltpu.async_copy(
                    tab4.at[gidx_v.at[cb + _RING]], slabs[b], sems.at[b]
                )

        return carry

    lax.fori_loop(0, _NCH // _RING, _group, 0)


def _extract_body(slab_ref, q_ref, o_ref):
    q = q_ref[...]
    acc = jnp.zeros((_BLK, _D), jnp.float32)
    for qq in range(4):
        part = slab_ref[:, qq * _D:(qq + 1) * _D]
        acc = acc + jnp.where(q == qq, part, 0.0)
    o_ref[...] = acc


_extract = pl.pallas_call(
    _extract_body,
    grid=(_N // _BLK,),
    in_specs=[
        pl.BlockSpec((_BLK, 4 * _D), lambda i: (i, 0)),
        pl.BlockSpec((_BLK, _D), lambda i: (i, 0)),
    ],
    out_specs=pl.BlockSpec((_BLK, _D), lambda i: (i, 0)),
    out_shape=jax.ShapeDtypeStruct((_N, _D), jnp.float32),
)


def kernel(inputs, table):
    prefix = jnp.arange(_F, dtype=jnp.int32) * _VOCAB
    fused = inputs + prefix[None, :]
    gidx = lax.shift_right_logical(fused, 2).reshape(_NW, _NCH, _CH)
    qb = jnp.broadcast_to(
        lax.bitwise_and(fused, 3).reshape(_N, 1), (_N, _D)
    )
    tab4 = table.reshape(-1, 4 * _D)
    slabs = _sc_gather(tab4, gidx)
    out = _extract(slabs, qb)
    return out.reshape(_B, _F, _D)


# speedup vs baseline: 1.0024x; 1.0024x over previous
import functools

import jax
import jax.numpy as jnp
from jax import lax
from jax.experimental import pallas as pl
from jax.experimental.pallas import tpu as pltpu
from jax.experimental.pallas import tpu_sc as plsc

_F = 26
_D = 32
_B = 4096
_VOCAB = 100000
_N = _B * _F

_NW = 32
_BPW = _N // _NW
_CH = 128
_NCH = _BPW // _CH
_RING = 6

_BLK = 1024

_mesh = plsc.VectorSubcoreMesh(core_axis_name="c", subcore_axis_name="s")


@functools.partial(
    pl.kernel,
    mesh=_mesh,
    out_type=jax.ShapeDtypeStruct((_N, 4 * _D), jnp.float32),
    scratch_types=[
        pltpu.VMEM((_NCH, _CH), jnp.int32),
    ]
    + [pltpu.VMEM((_CH, _CH), jnp.float32) for _ in range(_RING)]
    + [
        pltpu.SemaphoreType.DMA((_RING,)),
    ],
)
def _sc_gather(tab4, gidx_hbm, slab_out, gidx_v, *rest):
    slabs = list(rest[:_RING])
    sems = rest[_RING]
    w = lax.axis_index("s") * 2 + lax.axis_index("c")
    nbase = w * _BPW
    pltpu.sync_copy(gidx_hbm.at[w], gidx_v)

    for b in range(_RING):
        pltpu.async_copy(tab4.at[gidx_v.at[b]], slabs[b], sems.at[b])

    def _drain(cb, b):
        pltpu.make_async_copy(
            tab4.at[gidx_v.at[0]], slabs[b], sems.at[b]
        ).wait()
        pltpu.sync_copy(
            slabs[b], slab_out.at[pl.ds(nbase + cb * _CH, _CH)]
        )

    def _group(g, carry):
        for b in range(_RING):
            cb = g * _RING + b
            _drain(cb, b)

            @pl.when(cb + _RING < _NCH)
            def _():
                pltpu.async_copy(
                    tab4.at[gidx_v.at[cb + _RING]], slabs[b], sems.at[b]
                )

        return carry

    lax.fori_loop(0, _NCH // _RING, _group, 0)
    for cb in range(_NCH - _NCH % _RING, _NCH):
        _drain(cb, cb % _RING)


def _extract_body(slab_ref, q_ref, o_ref):
    q = q_ref[...]
    acc = jnp.zeros((_BLK, _D), jnp.float32)
    for qq in range(4):
        part = slab_ref[:, qq * _D:(qq + 1) * _D]
        acc = acc + jnp.where(q == qq, part, 0.0)
    o_ref[...] = acc


_extract = pl.pallas_call(
    _extract_body,
    grid=(_N // _BLK,),
    in_specs=[
        pl.BlockSpec((_BLK, 4 * _D), lambda i: (i, 0)),
        pl.BlockSpec((_BLK, _D), lambda i: (i, 0)),
    ],
    out_specs=pl.BlockSpec((_BLK, _D), lambda i: (i, 0)),
    out_shape=jax.ShapeDtypeStruct((_N, _D), jnp.float32),
)


def kernel(inputs, table):
    prefix = jnp.arange(_F, dtype=jnp.int32) * _VOCAB
    fused = inputs + prefix[None, :]
    gidx = lax.shift_right_logical(fused, 2).reshape(_NW, _NCH, _CH)
    qb = jnp.broadcast_to(
        lax.bitwise_and(fused, 3).reshape(_N, 1), (_N, _D)
    )
    tab4 = table.reshape(-1, 4 * _D)
    slabs = _sc_gather(tab4, gidx)
    out = _extract(slabs, qb)
    return out.reshape(_B, _F, _D)
